# baseline (speedup 1.0000x reference)
import functools
import math

import jax
import jax.numpy as jnp
import numpy as np
from jax import lax
from jax.experimental import pallas as pl
from jax.experimental.pallas import tpu as pltpu
from jax.experimental.pallas import tpu_sc as plsc

_NBUF = 4
_SEQ_PAD = 8


def _pe_table(seq_len: int, d_model: int) -> np.ndarray:
    pos = np.arange(seq_len)[:, None].astype(np.float32)
    div = np.exp(
        np.arange(0, d_model, 2).astype(np.float32) * -(math.log(10000.0) / d_model)
    )
    pe = np.zeros((seq_len, d_model), dtype=np.float32)
    pe[:, 0::2] = np.sin(pos * div)
    pe[:, 1::2] = np.cos(pos * div)
    return pe


@functools.cache
def _build_gather(batch: int, seq: int, vocab: int, d: int):
    n = batch * seq
    nc, ns, lanes = 2, 16, 16
    nw = nc * ns
    seq_p = ((seq + _SEQ_PAD - 1) // _SEQ_PAD) * _SEQ_PAD
    assert n % nw == 0 and d % lanes == 0
    per = n // nw
    chunk = seq_p
    nchunks = per // seq
    assert nchunks % _NBUF == 0 and chunk <= 128
    nsteps = nchunks // _NBUF
    mesh = plsc.VectorSubcoreMesh(core_axis_name="c", subcore_axis_name="s")

    @functools.partial(
        pl.kernel,
        mesh=mesh,
        out_type=jax.ShapeDtypeStruct((batch, seq_p, d), jnp.float32),
        scratch_types=[
            pltpu.VMEM((nchunks, chunk), jnp.int32),
        ]
        + [pltpu.VMEM((chunk, d), jnp.float32)] * _NBUF
        + [pltpu.SemaphoreType.DMA] * (2 * _NBUF),
    )
    def gather(table, idx, out, idx_v, *bufs):
        gbufs = bufs[:_NBUF]
        gsems = bufs[_NBUF : 2 * _NBUF]
        osems = bufs[2 * _NBUF :]
        wid = lax.axis_index("s") * nc + lax.axis_index("c")
        pltpu.sync_copy(idx.at[wid], idx_v)
        base_b = wid * nchunks

        for b in range(_NBUF):
            pltpu.async_copy(table.at[idx_v.at[b]], gbufs[b], gsems[b])

        def step(t, carry):
            j0 = t * _NBUF
            for b in range(_NBUF):
                pltpu.make_async_copy(
                    table.at[idx_v.at[0]], gbufs[b], gsems[b]
                ).wait()
                pltpu.async_copy(gbufs[b], out.at[base_b + j0 + b], osems[b])

            @pl.when(t < nsteps - 1)
            def _():
                for b in range(_NBUF):
                    pltpu.make_async_copy(gbufs[b], out.at[0], osems[b]).wait()
                    pltpu.async_copy(
                        table.at[idx_v.at[j0 + _NBUF + b]], gbufs[b], gsems[b]
                    )
            return carry

        lax.fori_loop(0, nsteps, step, 0)
        for b in range(_NBUF):
            pltpu.make_async_copy(gbufs[b], out.at[0], osems[b]).wait()

    return gather, nw, nchunks, chunk, seq_p


@functools.cache
def _build_epilogue(batch: int, seq: int, d: int, seq_p: int, bb: int):
    scale = np.float32(np.sqrt(np.float32(d)))

    def body(g_ref, pe_ref, o_ref):
        o_ref[...] = g_ref[...] * scale + pe_ref[...][None]

    return pl.pallas_call(
        body,
        grid=(batch // bb, seq_p // _SEQ_PAD),
        in_specs=[
            pl.BlockSpec((bb, _SEQ_PAD, d), lambda i, j: (i, j, 0)),
            pl.BlockSpec((_SEQ_PAD, d), lambda i, j: (j, 0)),
        ],
        out_specs=pl.BlockSpec((bb, _SEQ_PAD, d), lambda i, j: (i, j, 0)),
        out_shape=jax.ShapeDtypeStruct((batch, seq, d), jnp.float32),
    )


def kernel(text_ids, emb_table):
    batch, seq = text_ids.shape
    vocab, d = emb_table.shape
    gather, nw, nchunks, chunk, seq_p = _build_gather(batch, seq, vocab, d)
    assert chunk == seq_p
    epilogue = _build_epilogue(batch, seq, d, seq_p, 8)
    pe_np = np.zeros((seq_p, d), dtype=np.float32)
    pe_np[:seq] = _pe_table(seq, d)
    pe = jnp.asarray(pe_np)
    idx = text_ids.astype(jnp.int32).reshape(nw, nchunks, seq)
    idx = jnp.concatenate([idx, idx[:, :, : seq_p - seq]], axis=-1)
    gathered = gather(emb_table, idx)
    return epilogue(gathered, pe)


# speedup vs baseline: 5.7509x; 5.7509x over previous
import functools
import math

import jax
import jax.numpy as jnp
import numpy as np
from jax import lax
from jax.experimental import pallas as pl
from jax.experimental.pallas import tpu as pltpu
from jax.experimental.pallas import tpu_sc as plsc

_NBUF = 4
_SEQ_PAD = 8


def _pe_table(seq_len: int, d_model: int) -> np.ndarray:
    pos = np.arange(seq_len)[:, None].astype(np.float32)
    div = np.exp(
        np.arange(0, d_model, 2).astype(np.float32) * -(math.log(10000.0) / d_model)
    )
    pe = np.zeros((seq_len, d_model), dtype=np.float32)
    pe[:, 0::2] = np.sin(pos * div)
    pe[:, 1::2] = np.cos(pos * div)
    return pe


@functools.cache
def _build_gather(batch: int, seq: int, vocab: int, d: int):
    n = batch * seq
    nc, ns, lanes = 2, 16, 16
    nw = nc * ns
    seq_p = ((seq + _SEQ_PAD - 1) // _SEQ_PAD) * _SEQ_PAD
    assert n % nw == 0 and d % lanes == 0
    per = n // nw
    chunk = seq_p
    nchunks = per // seq
    assert nchunks % _NBUF == 0 and chunk <= 128
    nsteps = nchunks // _NBUF
    mesh = plsc.VectorSubcoreMesh(core_axis_name="c", subcore_axis_name="s")

    @functools.partial(
        pl.kernel,
        mesh=mesh,
        out_type=jax.ShapeDtypeStruct((batch, seq_p, d), jnp.float32),
        scratch_types=[
            pltpu.VMEM((nchunks, chunk), jnp.int32),
        ]
        + [pltpu.VMEM((chunk, d), jnp.float32)] * _NBUF
        + [pltpu.SemaphoreType.DMA] * (2 * _NBUF),
    )
    def gather(table, idx, out, idx_v, *bufs):
        gbufs = bufs[:_NBUF]
        gsems = bufs[_NBUF : 2 * _NBUF]
        osems = bufs[2 * _NBUF :]
        wid = lax.axis_index("s") * nc + lax.axis_index("c")
        pltpu.sync_copy(idx.at[wid], idx_v)
        base_b = wid * nchunks

        for b in range(_NBUF):
            pltpu.async_copy(table.at[idx_v.at[b]], gbufs[b], gsems[b])

        def step(t, carry):
            j0 = t * _NBUF
            for b in range(_NBUF):
                pltpu.make_async_copy(
                    table.at[idx_v.at[0]], gbufs[b], gsems[b]
                ).wait()
                pltpu.async_copy(gbufs[b], out.at[base_b + j0 + b], osems[b])

            @pl.when(t < nsteps - 1)
            def _():
                for b in range(_NBUF):
                    pltpu.make_async_copy(gbufs[b], out.at[0], osems[b]).wait()
                    pltpu.async_copy(
                        table.at[idx_v.at[j0 + _NBUF + b]], gbufs[b], gsems[b]
                    )
            return carry

        lax.fori_loop(0, nsteps, step, 0)
        for b in range(_NBUF):
            pltpu.make_async_copy(gbufs[b], out.at[0], osems[b]).wait()

    return gather, nw, nchunks, chunk, seq_p


@functools.cache
def _build_epilogue(batch: int, seq: int, d: int, seq_p: int, bb: int):
    scale = np.float32(np.sqrt(np.float32(d)))

    def body(g_ref, pe_ref, o_ref):
        o_ref[...] = g_ref[:, :seq, :] * scale + pe_ref[:, :seq, :]

    return pl.pallas_call(
        body,
        grid=(batch // bb,),
        in_specs=[
            pl.BlockSpec((bb, seq_p, d), lambda i: (i, 0, 0)),
            pl.BlockSpec((bb, seq_p, d), lambda i: (0, 0, 0)),
        ],
        out_specs=pl.BlockSpec((bb, seq, d), lambda i: (i, 0, 0)),
        out_shape=jax.ShapeDtypeStruct((batch, seq, d), jnp.float32),
    )


def kernel(text_ids, emb_table):
    batch, seq = text_ids.shape
    vocab, d = emb_table.shape
    gather, nw, nchunks, chunk, seq_p = _build_gather(batch, seq, vocab, d)
    assert chunk == seq_p
    bb = 64
    epilogue = _build_epilogue(batch, seq, d, seq_p, bb)
    pe_np = np.zeros((seq_p, d), dtype=np.float32)
    pe_np[:seq] = _pe_table(seq, d)
    pe = jnp.asarray(np.tile(pe_np, (bb, 1, 1)))
    idx = text_ids.astype(jnp.int32).reshape(nw, nchunks, seq)
    idx = jnp.concatenate([idx, idx[:, :, : seq_p - seq]], axis=-1)
    gathered = gather(emb_table, idx)
    return epilogue(gathered, pe)


# speedup vs baseline: 7.7723x; 1.3515x over previous
import functools
import math

import jax
import jax.numpy as jnp
import numpy as np
from jax import lax
from jax.experimental import pallas as pl
from jax.experimental.pallas import tpu as pltpu
from jax.experimental.pallas import tpu_sc as plsc

_NBUF = 4


def _pe_table(seq_len: int, d_model: int) -> np.ndarray:
    pos = np.arange(seq_len)[:, None].astype(np.float32)
    div = np.exp(
        np.arange(0, d_model, 2).astype(np.float32) * -(math.log(10000.0) / d_model)
    )
    pe = np.zeros((seq_len, d_model), dtype=np.float32)
    pe[:, 0::2] = np.sin(pos * div)
    pe[:, 1::2] = np.cos(pos * div)
    return pe


@functools.cache
def _build_gather_scatter(batch: int, seq: int, vocab: int, d: int):
    n = batch * seq
    nc, ns, lanes = 2, 16, 16
    nw = nc * ns
    assert n % nw == 0 and d % lanes == 0
    per = n // nw
    chunk = seq
    nchunks = per // chunk
    assert nchunks % _NBUF == 0 and chunk <= 128
    nsteps = nchunks // _NBUF
    mesh = plsc.VectorSubcoreMesh(core_axis_name="c", subcore_axis_name="s")

    @functools.partial(
        pl.kernel,
        mesh=mesh,
        out_type=jax.ShapeDtypeStruct((n, d), jnp.float32),
        scratch_types=[
            pltpu.VMEM((nchunks, chunk), jnp.int32),
            pltpu.VMEM((nchunks, chunk), jnp.int32),
        ]
        + [pltpu.VMEM((chunk, d), jnp.float32)] * _NBUF
        + [pltpu.SemaphoreType.DMA] * (2 * _NBUF),
    )
    def gather_scatter(table, idx, oidx, out, idx_v, oidx_v, *bufs):
        gbufs = bufs[:_NBUF]
        gsems = bufs[_NBUF : 2 * _NBUF]
        osems = bufs[2 * _NBUF :]
        wid = lax.axis_index("s") * nc + lax.axis_index("c")
        pltpu.sync_copy(idx.at[wid], idx_v)
        pltpu.sync_copy(oidx.at[wid], oidx_v)

        for b in range(_NBUF):
            pltpu.async_copy(table.at[idx_v.at[b]], gbufs[b], gsems[b])

        def step(t, carry):
            j0 = t * _NBUF
            for b in range(_NBUF):
                pltpu.make_async_copy(
                    table.at[idx_v.at[0]], gbufs[b], gsems[b]
                ).wait()
                pltpu.async_copy(gbufs[b], out.at[oidx_v.at[j0 + b]], osems[b])

            @pl.when(t < nsteps - 1)
            def _():
                for b in range(_NBUF):
                    pltpu.make_async_copy(
                        gbufs[b], out.at[oidx_v.at[0]], osems[b]
                    ).wait()
                    pltpu.async_copy(
                        table.at[idx_v.at[j0 + _NBUF + b]], gbufs[b], gsems[b]
                    )
            return carry

        lax.fori_loop(0, nsteps, step, 0)
        for b in range(_NBUF):
            pltpu.make_async_copy(
                gbufs[b], out.at[oidx_v.at[0]], osems[b]
            ).wait()

    return gather_scatter, nw, nchunks, chunk


@functools.cache
def _build_epilogue(batch: int, seq: int, d: int, bb: int):
    scale = np.float32(np.sqrt(np.float32(d)))

    def body(g_ref, pe_ref, o_ref):
        o_ref[...] = g_ref[...] * scale + pe_ref[...]

    return pl.pallas_call(
        body,
        grid=(batch // bb,),
        in_specs=[
            pl.BlockSpec((seq, bb, d), lambda i: (0, i, 0)),
            pl.BlockSpec((seq, bb, d), lambda i: (0, 0, 0)),
        ],
        out_specs=pl.BlockSpec((seq, bb, d), lambda i: (0, i, 0)),
        out_shape=jax.ShapeDtypeStruct((seq, batch, d), jnp.float32),
    )


def kernel(text_ids, emb_table):
    batch, seq = text_ids.shape
    vocab, d = emb_table.shape
    gather_scatter, nw, nchunks, chunk = _build_gather_scatter(batch, seq, vocab, d)
    bb = 64
    epilogue = _build_epilogue(batch, seq, d, bb)
    pe_rep = jnp.asarray(np.tile(_pe_table(seq, d)[:, None, :], (1, bb, 1)))
    idx = text_ids.astype(jnp.int32).reshape(nw, nchunks, seq)
    brow = np.arange(batch, dtype=np.int32).reshape(nw, nchunks, 1)
    oidx = jnp.asarray(brow + batch * np.arange(seq, dtype=np.int32)[None, None, :])
    inter = gather_scatter(emb_table, idx, oidx)
    out_t = epilogue(inter.reshape(seq, batch, d), pe_rep)
    return out_t.transpose(1, 0, 2)
